# Initial kernel scaffold; baseline (speedup 1.0000x reference)
import jax
import jax.numpy as jnp
from jax.experimental import pallas as pl


def kernel(x, edge_index, batch, undirected_edge_mask, l2_node_mapping, l2_edge_index, num_graphs, W_gcn1, b_gcn1, W_ne0, b_ne0, g_ne, bt_ne, W_ne1, b_ne1, W_ef0, b_ef0, g_ef, bt_ef, W_ef1, b_ef1, g_bn1, bt_bn1, g_bn2, bt_bn2, W_gcn2, b_gcn2, W_f0, b_f0, g_ln0, bt_ln0, W_f1, b_f1, g_ln1, bt_ln1, W_f2, b_f2):
    raise NotImplementedError("write your pallas kernel here")


# speedup vs baseline: 4.3455x; 4.3455x over previous
import functools

import jax
import jax.numpy as jnp
from jax.experimental import pallas as pl
from jax.experimental.pallas import tpu as pltpu

EPS = 1e-5


def _gelu(x):
    return 0.5 * x * (1.0 + jax.lax.erf(x * 0.7071067811865476))


def _rows_spec(blk, w):
    return pl.BlockSpec((blk, w), lambda i: (i, 0))


def _stats_spec():
    return pl.BlockSpec((8, 128), lambda i: (0, 0))


def _stats_merge(st_ref, r, x, blk):
    i = pl.program_id(0)
    mb = jnp.sum(x, axis=0, keepdims=True) * (1.0 / blk)
    m2b = jnp.sum((x - mb) ** 2, axis=0, keepdims=True)

    @pl.when(i == 0)
    def _():
        st_ref[r:r + 1] = mb
        st_ref[r + 1:r + 2] = m2b

    @pl.when(i != 0)
    def _():
        nf = i.astype(jnp.float32) * float(blk)
        mean = st_ref[r:r + 1]
        delta = mb - mean
        tot = nf + float(blk)
        st_ref[r:r + 1] = mean + delta * (float(blk) / tot)
        st_ref[r + 1:r + 2] = (st_ref[r + 1:r + 2] + m2b
                               + delta * delta * (nf * float(blk) / tot))


def _scale_rows_matmul(x_ref, w_ref, s_ref, o_ref):
    h = jnp.dot(x_ref[...], w_ref[...], preferred_element_type=jnp.float32)
    o_ref[...] = s_ref[...] * h


def tc_scale_rows_matmul(x, w, s, blk):
    n, d = x.shape
    return pl.pallas_call(
        _scale_rows_matmul,
        grid=(n // blk,),
        in_specs=[_rows_spec(blk, d), pl.BlockSpec((d, w.shape[1]), lambda i: (0, 0)),
                  _rows_spec(blk, 1)],
        out_specs=_rows_spec(blk, w.shape[1]),
        out_shape=jax.ShapeDtypeStruct((n, w.shape[1]), jnp.float32),
    )(x, w, s)


def _gcn_post(agg_ref, hn_ref, s_ref, b_ref, o_ref):
    o_ref[...] = s_ref[...] * (agg_ref[...] + hn_ref[...]) + b_ref[...]


def tc_gcn_post(agg, hn, s, b, blk):
    n, d = agg.shape
    return pl.pallas_call(
        _gcn_post,
        grid=(n // blk,),
        in_specs=[_rows_spec(blk, d), _rows_spec(blk, d), _rows_spec(blk, 1),
                  pl.BlockSpec((1, d), lambda i: (0, 0))],
        out_specs=_rows_spec(blk, d),
        out_shape=jax.ShapeDtypeStruct((n, d), jnp.float32),
    )(agg, hn, s, b.reshape(1, d))


def _p1_body(blk, hu_ref, hv_ref, w_ref, b_ref, u_ref, es_ref, st_ref):
    hu = hu_ref[...]
    hv = hv_ref[...]
    w = w_ref[...]
    b = b_ref[...]
    c1 = jnp.concatenate([hu, hv], axis=1)
    c2 = jnp.concatenate([hv, hu], axis=1)
    u1 = jnp.dot(c1, w, preferred_element_type=jnp.float32) + b
    u2 = jnp.dot(c2, w, preferred_element_type=jnp.float32) + b
    u = jnp.concatenate([u1, u2], axis=1)
    u_ref[...] = u
    es_ref[...] = hu + hv
    _stats_merge(st_ref, 0, u, blk)


def tc_p1(hu, hv, w_ne0, b_ne0, blk):
    e = hu.shape[0]
    return pl.pallas_call(
        functools.partial(_p1_body, float(blk)),
        grid=(e // blk,),
        in_specs=[_rows_spec(blk, 128), _rows_spec(blk, 128),
                  pl.BlockSpec((256, 64), lambda i: (0, 0)),
                  pl.BlockSpec((1, 64), lambda i: (0, 0))],
        out_specs=[_rows_spec(blk, 128), _rows_spec(blk, 128), _stats_spec()],
        out_shape=[jax.ShapeDtypeStruct((e, 128), jnp.float32),
                   jax.ShapeDtypeStruct((e, 128), jnp.float32),
                   jax.ShapeDtypeStruct((8, 128), jnp.float32)],
    )(hu, hv, w_ne0, b_ne0.reshape(1, 64))


def _p2_body(u_ref, s_ref, t_ref, w_ref, b_ref, o_ref):
    g = _gelu(u_ref[...] * s_ref[...] + t_ref[...])
    w = w_ref[...]
    b = b_ref[...]
    o1 = jnp.dot(g[:, :64], w, preferred_element_type=jnp.float32) + b
    o2 = jnp.dot(g[:, 64:], w, preferred_element_type=jnp.float32) + b
    o_ref[...] = 0.5 * (o1 + o2)


def tc_p2(u, s, t, w_ne1, b_ne1, blk):
    e = u.shape[0]
    return pl.pallas_call(
        _p2_body,
        grid=(e // blk,),
        in_specs=[_rows_spec(blk, 128),
                  pl.BlockSpec((1, 128), lambda i: (0, 0)),
                  pl.BlockSpec((1, 128), lambda i: (0, 0)),
                  pl.BlockSpec((64, 64), lambda i: (0, 0)),
                  pl.BlockSpec((1, 64), lambda i: (0, 0))],
        out_specs=_rows_spec(blk, 64),
        out_shape=jax.ShapeDtypeStruct((e, 64), jnp.float32),
    )(u, s.reshape(1, 128), t.reshape(1, 128), w_ne1, b_ne1.reshape(1, 64))


def _p3_body(blk, he_ref, hf_ref, ee_ref, ef_ref, w_ref, b_ref,
             v_ref, hm_ref, st_ref):
    he = he_ref[...]
    hf = hf_ref[...]
    w = w_ref[...]
    b = b_ref[...]
    v1 = jnp.dot(jnp.concatenate([hf, he], axis=1), w,
                 preferred_element_type=jnp.float32) + b
    v2 = jnp.dot(jnp.concatenate([he, hf], axis=1), w,
                 preferred_element_type=jnp.float32) + b
    v_ref[:, :128] = v1
    v_ref[:, 128:] = v2
    hm_ref[...] = 0.25 * (ee_ref[...] + ef_ref[...])
    _stats_merge(st_ref, 0, v1, blk)
    _stats_merge(st_ref, 2, v2, blk)


def tc_p3(he, hf, ee, ef, w_ef0, b_ef0, blk):
    m = he.shape[0]
    return pl.pallas_call(
        functools.partial(_p3_body, float(blk)),
        grid=(m // blk,),
        in_specs=[_rows_spec(blk, 64), _rows_spec(blk, 64),
                  _rows_spec(blk, 128), _rows_spec(blk, 128),
                  pl.BlockSpec((128, 128), lambda i: (0, 0)),
                  pl.BlockSpec((1, 128), lambda i: (0, 0))],
        out_specs=[_rows_spec(blk, 256), _rows_spec(blk, 128), _stats_spec()],
        out_shape=[jax.ShapeDtypeStruct((m, 256), jnp.float32),
                   jax.ShapeDtypeStruct((m, 128), jnp.float32),
                   jax.ShapeDtypeStruct((8, 128), jnp.float32)],
    )(he, hf, ee, ef, w_ef0, b_ef0.reshape(1, 128))


def _p4_body(blk, v_ref, s_ref, t_ref, w_ref, b_ref, h1_ref, st_ref):
    s = s_ref[...]
    t = t_ref[...]
    g1 = _gelu(v_ref[:, :128] * s[:, :128] + t[:, :128])
    g2 = _gelu(v_ref[:, 128:] * s[:, 128:] + t[:, 128:])
    w = w_ref[...]
    b = b_ref[...]
    h1a = jnp.dot(g1, w, preferred_element_type=jnp.float32) + b
    h1b = jnp.dot(g2, w, preferred_element_type=jnp.float32) + b
    h1 = 0.5 * (h1a + h1b)
    h1_ref[...] = h1
    _stats_merge(st_ref, 0, h1, blk)


def tc_p4(v, s, t, w_ef1, b_ef1, blk):
    m = v.shape[0]
    return pl.pallas_call(
        functools.partial(_p4_body, float(blk)),
        grid=(m // blk,),
        in_specs=[_rows_spec(blk, 256),
                  pl.BlockSpec((1, 256), lambda i: (0, 0)),
                  pl.BlockSpec((1, 256), lambda i: (0, 0)),
                  pl.BlockSpec((128, 128), lambda i: (0, 0)),
                  pl.BlockSpec((1, 128), lambda i: (0, 0))],
        out_specs=[_rows_spec(blk, 128), _stats_spec()],
        out_shape=[jax.ShapeDtypeStruct((m, 128), jnp.float32),
                   jax.ShapeDtypeStruct((8, 128), jnp.float32)],
    )(v, s.reshape(1, 256), t.reshape(1, 256), w_ef1, b_ef1.reshape(1, 128))


def _p5_body(h1_ref, hm_ref, s_ref, t_ref, dinv_ref, w_ref, hn_ref, h2n_ref):
    h1n = hm_ref[...] + jax.nn.relu(h1_ref[...] * s_ref[...] + t_ref[...])
    hn_ref[...] = h1n
    h2 = jnp.dot(h1n, w_ref[...], preferred_element_type=jnp.float32)
    h2n_ref[...] = dinv_ref[...] * h2


def tc_p5(h1, hm, s, t, dinv2, w_gcn2, blk):
    m = h1.shape[0]
    return pl.pallas_call(
        _p5_body,
        grid=(m // blk,),
        in_specs=[_rows_spec(blk, 128), _rows_spec(blk, 128),
                  pl.BlockSpec((1, 128), lambda i: (0, 0)),
                  pl.BlockSpec((1, 128), lambda i: (0, 0)),
                  _rows_spec(blk, 1),
                  pl.BlockSpec((128, 128), lambda i: (0, 0))],
        out_specs=[_rows_spec(blk, 128), _rows_spec(blk, 128)],
        out_shape=[jax.ShapeDtypeStruct((m, 128), jnp.float32),
                   jax.ShapeDtypeStruct((m, 128), jnp.float32)],
    )(h1, hm, s.reshape(1, 128), t.reshape(1, 128), dinv2, w_gcn2)


def _p6_body(blk, agg_ref, h2n_ref, dinv_ref, b_ref, h2_ref, st_ref):
    h2 = dinv_ref[...] * (agg_ref[...] + h2n_ref[...]) + b_ref[...]
    h2_ref[...] = h2
    _stats_merge(st_ref, 0, h2, blk)


def tc_p6(agg, h2n, dinv2, b_gcn2, blk):
    m = agg.shape[0]
    return pl.pallas_call(
        functools.partial(_p6_body, float(blk)),
        grid=(m // blk,),
        in_specs=[_rows_spec(blk, 128), _rows_spec(blk, 128), _rows_spec(blk, 1),
                  pl.BlockSpec((1, 128), lambda i: (0, 0))],
        out_specs=[_rows_spec(blk, 128), _stats_spec()],
        out_shape=[jax.ShapeDtypeStruct((m, 128), jnp.float32),
                   jax.ShapeDtypeStruct((8, 128), jnp.float32)],
    )(agg, h2n, dinv2, b_gcn2.reshape(1, 128))


def _p7_body(h2_ref, hn_ref, s_ref, t_ref, o_ref):
    o_ref[...] = hn_ref[...] + jax.nn.relu(h2_ref[...] * s_ref[...] + t_ref[...])


def tc_p7(h2, h1n, s, t, blk):
    m = h2.shape[0]
    return pl.pallas_call(
        _p7_body,
        grid=(m // blk,),
        in_specs=[_rows_spec(blk, 128), _rows_spec(blk, 128),
                  pl.BlockSpec((1, 128), lambda i: (0, 0)),
                  pl.BlockSpec((1, 128), lambda i: (0, 0))],
        out_specs=_rows_spec(blk, 128),
        out_shape=jax.ShapeDtypeStruct((m, 128), jnp.float32),
    )(h2, h1n, s.reshape(1, 128), t.reshape(1, 128))


def _ln(x, g, b):
    mu = jnp.mean(x, axis=-1, keepdims=True)
    var = jnp.mean((x - mu) ** 2, axis=-1, keepdims=True)
    return (x - mu) / jnp.sqrt(var + EPS) * g + b


def _p8_body(sums_ref, cnt_ref, w0_ref, b0_ref, g0_ref, t0_ref,
             w1_ref, b1_ref, g1_ref, t1_ref, w2_ref, b2_ref, o_ref):
    hp = sums_ref[...] / jnp.maximum(cnt_ref[...], 1.0)
    h = _gelu(_ln(jnp.dot(hp, w0_ref[...], preferred_element_type=jnp.float32)
                  + b0_ref[...], g0_ref[...], t0_ref[...]))
    h = _gelu(_ln(jnp.dot(h, w1_ref[...], preferred_element_type=jnp.float32)
                  + b1_ref[...], g1_ref[...], t1_ref[...]))
    o_ref[...] = jnp.dot(h, w2_ref[...], preferred_element_type=jnp.float32) \
        + b2_ref[...]


def tc_p8(sums, cnt, w_f0, b_f0, g_ln0, t_ln0, w_f1, b_f1, g_ln1, t_ln1,
          w_f2, b_f2):
    full = lambda shape: pl.BlockSpec(shape, lambda: (0,) * len(shape))
    return pl.pallas_call(
        _p8_body,
        in_specs=[full((256, 128)), full((256, 1)),
                  full((128, 128)), full((1, 128)), full((1, 128)), full((1, 128)),
                  full((128, 128)), full((1, 128)), full((1, 128)), full((1, 128)),
                  full((128, 1)), full((1, 1))],
        out_specs=full((256, 1)),
        out_shape=jax.ShapeDtypeStruct((256, 1), jnp.float32),
    )(sums, cnt.reshape(256, 1), w_f0, b_f0.reshape(1, 128),
      g_ln0.reshape(1, 128), t_ln0.reshape(1, 128), w_f1, b_f1.reshape(1, 128),
      g_ln1.reshape(1, 128), t_ln1.reshape(1, 128), w_f2, b_f2.reshape(1, 1))


def _bn_affine(mu, m2, n, g, b):
    var = m2 / n
    s = g / jnp.sqrt(var + EPS)
    return s, b - mu * s


def kernel(x, edge_index, batch, undirected_edge_mask, l2_node_mapping,
           l2_edge_index, num_graphs, W_gcn1, b_gcn1, W_ne0, b_ne0, g_ne,
           bt_ne, W_ne1, b_ne1, W_ef0, b_ef0, g_ef, bt_ef, W_ef1, b_ef1,
           g_bn1, bt_bn1, g_bn2, bt_bn2, W_gcn2, b_gcn2, W_f0, b_f0, g_ln0,
           bt_ln0, W_f1, b_f1, g_ln1, bt_ln1, W_f2, b_f2):
    n = x.shape[0]
    e = edge_index.shape[1]
    m = l2_node_mapping.shape[1]
    src, dst = edge_index[0], edge_index[1]

    deg1 = jnp.zeros((n,), jnp.float32).at[dst].add(1.0) + 1.0
    dinv1 = jax.lax.rsqrt(deg1).reshape(n, 1)
    h1n = tc_scale_rows_matmul(x, W_gcn1, dinv1, 2000)
    agg1 = jnp.zeros((n, 128), jnp.float32).at[dst].add(h1n[src])
    H0 = tc_gcn_post(agg1, h1n, dinv1, b_gcn1, 2000)

    Hu = H0[src]
    Hv = H0[dst]
    U, Esum, st1 = tc_p1(Hu, Hv, W_ne0, b_ne0, 4000)
    g2 = jnp.concatenate([g_ne, g_ne])
    b2 = jnp.concatenate([bt_ne, bt_ne])
    s_ne, t_ne = _bn_affine(st1[0], st1[1], float(e), g2, b2)
    h_edge = tc_p2(U, s_ne, t_ne, W_ne1, b_ne1, 4000)

    e_idx = l2_node_mapping[0]
    f_idx = l2_node_mapping[1]
    he = h_edge[e_idx]
    hf = h_edge[f_idx]
    ee = Esum[e_idx]
    ef = Esum[f_idx]
    V, H0m, st3 = tc_p3(he, hf, ee, ef, W_ef0, b_ef0, 4000)
    s_ef1, t_ef1 = _bn_affine(st3[0], st3[1], float(m), g_ef, bt_ef)
    s_ef2, t_ef2 = _bn_affine(st3[2], st3[3], float(m), g_ef, bt_ef)
    s_ef = jnp.concatenate([s_ef1, s_ef2])
    t_ef = jnp.concatenate([t_ef1, t_ef2])
    H1, st4 = tc_p4(V, s_ef, t_ef, W_ef1, b_ef1, 4000)
    s_b1, t_b1 = _bn_affine(st4[0], st4[1], float(m), g_bn1, bt_bn1)

    src2, dst2 = l2_edge_index[0], l2_edge_index[1]
    deg2 = jnp.zeros((m,), jnp.float32).at[dst2].add(1.0) + 1.0
    dinv2 = jax.lax.rsqrt(deg2).reshape(m, 1)
    H1_new, h2n = tc_p5(H1, H0m, s_b1, t_b1, dinv2, W_gcn2, 4000)
    agg2 = jnp.zeros((m, 128), jnp.float32).at[dst2].add(h2n[src2])
    H2, st6 = tc_p6(agg2, h2n, dinv2, b_gcn2, 4000)
    s_b2, t_b2 = _bn_affine(st6[0], st6[1], float(m), g_bn2, bt_bn2)
    H2_new = tc_p7(H2, H1_new, s_b2, t_b2, 4000)

    l2_batch = batch[src[e_idx]]
    sums = jnp.zeros((256, 128), jnp.float32).at[l2_batch].add(H2_new)
    cnt = jnp.zeros((256,), jnp.float32).at[l2_batch].add(1.0)
    return tc_p8(sums, cnt, W_f0, b_f0, g_ln0, bt_ln0, W_f1, b_f1,
                 g_ln1, bt_ln1, W_f2, b_f2)
